# Initial kernel scaffold; baseline (speedup 1.0000x reference)
import jax
import jax.numpy as jnp
from jax.experimental import pallas as pl


def kernel(x_shop, x_public, edge_index_ss, edge_index_sp, edge_index_ps, edge_index_pp, params):
    raise NotImplementedError("write your pallas kernel here")


# speedup vs baseline: 6.3636x; 6.3636x over previous
import jax
import jax.numpy as jnp
from jax import lax
from jax.experimental import pallas as pl
from jax.experimental.pallas import tpu as pltpu, tpu_sc as plsc

N = 50000
E = 800000
F32 = jnp.float32

NC, NS = 2, 16
NPAD = 50048
RPT = NPAD // NS
ZROWS = 136
EPAD = 819200
NROWS = EPAD // 128
C1 = 8
C2 = 4
L1_ROWS = NROWS // (NC * NS)
L2_ROWS = NROWS // NS


def _sc_mesh():
    return plsc.VectorSubcoreMesh(
        core_axis_name="c", subcore_axis_name="s", num_cores=NC, num_subcores=NS
    )


_SC_PARAMS = pltpu.CompilerParams(use_tc_tiling_on_sc=False)


def _zero_zbuf(zbuf, width):
    @pl.loop(0, ZROWS)
    def _(i):
        for w in range(width // 16):
            zbuf[i, pl.ds(w * 16, 16)] = jnp.zeros((16,), F32)


def _segment_pass(tbl, src2d, dst2d, out, acc, idxs, idxd, rows, zbuf,
                  bounce, sem, c, s, base_rows, cc):
    my_acc_row = s * RPT

    @pl.loop(0, RPT // ZROWS)
    def _(i):
        pltpu.sync_copy(zbuf, acc.at[pl.ds(my_acc_row + i * ZROWS, ZROWS)])

    plsc.subcore_barrier()

    n_chunks = base_rows[2] // cc

    @pl.loop(0, n_chunks)
    def _(k):
        rs = base_rows[0] + k * cc
        rd = base_rows[1] + k * cc
        pltpu.sync_copy(src2d.at[pl.ds(rs, cc)], idxs)
        pltpu.sync_copy(dst2d.at[pl.ds(rd, cc)], idxd)
        descs = [pltpu.async_copy(tbl.at[idxs.at[j]], rows.at[j], sem)
                 for j in range(cc)]
        for d in descs:
            d.wait()
        for j in range(cc):
            pltpu.sync_copy(rows.at[j], acc.at[idxd.at[j]], add=True)

    plsc.subcore_barrier()

    @pl.loop(0, RPT // ZROWS)
    def _(i):
        r = my_acc_row + i * ZROWS
        pltpu.sync_copy(acc.at[pl.ds(r, ZROWS)], bounce)
        pltpu.sync_copy(bounce, out.at[pl.ds(c * NPAD + r, ZROWS)])

    plsc.subcore_barrier()


def _sc_l1_body(xs, xp, s_ss, d_ss, s_sp, d_sp, s_ps, d_ps, s_pp, d_pp,
                o_ss, o_sp, o_ps, o_pp, acc, idxs, idxd, rows, zbuf, bounce,
                sem):
    c = lax.axis_index("c")
    s = lax.axis_index("s")
    _zero_zbuf(zbuf, 16)
    base = c * (NROWS // NC) + s * L1_ROWS
    for tbl, src2d, dst2d, out in (
        (xs, s_ss, d_ss, o_ss), (xs, s_sp, d_sp, o_sp),
        (xp, s_ps, d_ps, o_ps), (xp, s_pp, d_pp, o_pp),
    ):
        _segment_pass(tbl, src2d, dst2d, out, acc, idxs, idxd, rows, zbuf,
                      bounce, sem, c, s, (base, base, L1_ROWS), C1)


def _sc_l2_body(hs, hp, s_ss, d_ss, s_sp, d_sp, s_ps, d_ps, s_pp, d_pp,
                o_ss, o_sp, o_ps, o_pp, acc, idxs, idxd, rows, zbuf, bounce,
                sem):
    c = lax.axis_index("c")
    s = lax.axis_index("s")
    _zero_zbuf(zbuf, 32)
    base_s = c * NROWS + s * L2_ROWS
    base_d = s * L2_ROWS
    for tbl, src2d, dst2d, out in (
        (hs, s_ss, d_ss, o_ss), (hs, s_sp, d_sp, o_sp),
        (hp, s_ps, d_ps, o_ps), (hp, s_pp, d_pp, o_pp),
    ):
        _segment_pass(tbl, src2d, dst2d, out, acc, idxs, idxd, rows, zbuf,
                      bounce, sem, c, s, (base_s, base_d, L2_ROWS), C2)


def _dense1_body(p0a, p1a, p0b, p1b, x, wla, wlb, wra, wrb, bla, blb,
                 h_out, inv_out):
    sa = p0a[...] + p1a[...]
    sb = p0b[...] + p1b[...]
    inva = 1.0 / jnp.maximum(sa[:, 11:12], 1.0)
    invb = 1.0 / jnp.maximum(sb[:, 11:12], 1.0)
    ma = sa * inva
    mb = sb * invb
    h = (jnp.dot(ma, wla[...], preferred_element_type=F32)
         + jnp.dot(mb, wlb[...], preferred_element_type=F32)
         + jnp.dot(x[...], wra[...] + wrb[...], preferred_element_type=F32)
         + bla[...] + blb[...])
    h = jnp.maximum(h, 0.0)
    h_out[0] = h[:, :32]
    h_out[1] = h[:, 32:]
    inv_out[...] = jnp.concatenate(
        [inva, invb, jnp.zeros((inva.shape[0], 6), F32)], axis=1)


def _dense2_body(sa0, sa1, sb0, sb1, h0, h1, inv8, wla, wlb, wra, wrb,
                 bla, blb, lw, lb, out):
    ma = jnp.concatenate([sa0[...], sa1[...]], axis=1) * inv8[:, 0:1]
    mb = jnp.concatenate([sb0[...], sb1[...]], axis=1) * inv8[:, 1:2]
    hh = jnp.concatenate([h0[...], h1[...]], axis=1)
    h2 = (jnp.dot(ma, wla[...], preferred_element_type=F32)
          + jnp.dot(mb, wlb[...], preferred_element_type=F32)
          + jnp.dot(hh, wra[...] + wrb[...], preferred_element_type=F32)
          + bla[...] + blb[...])
    h2 = jnp.maximum(h2, 0.0)
    out[...] = jnp.dot(h2, lw[...], preferred_element_type=F32) + lb[...]


def _prep_edges(ei):
    src = jnp.concatenate([ei[0], jnp.zeros((EPAD - E,), jnp.int32)])
    dst = jnp.concatenate([ei[1], jnp.full((EPAD - E,), N, jnp.int32)])
    src2d = src.reshape(NROWS, 128)
    dst2d = dst.reshape(NROWS, 128)
    src_l2 = jnp.concatenate([src2d, src2d + N], axis=0)
    return src2d, dst2d, src_l2


def kernel(x_shop, x_public, edge_index_ss, edge_index_sp, edge_index_ps,
           edge_index_pp, params):
    one = jnp.ones((N, 1), F32)
    zpad = jnp.zeros((N, 4), F32)
    xs = jnp.concatenate([x_shop, one, zpad], axis=1)
    xp = jnp.concatenate([x_public, one, zpad], axis=1)

    ss = _prep_edges(edge_index_ss)
    sp = _prep_edges(edge_index_sp)
    ps = _prep_edges(edge_index_ps)
    pp = _prep_edges(edge_index_pp)

    wz = jnp.zeros((5, 64), F32)
    pad16 = lambda w: jnp.concatenate([w, wz], axis=0)
    p = params

    sc1 = pl.kernel(
        _sc_l1_body,
        out_type=[jax.ShapeDtypeStruct((NC * NPAD, 16), F32)] * 4,
        mesh=_sc_mesh(),
        scratch_types=[
            pltpu.VMEM_SHARED((NPAD, 16), F32),
            pltpu.VMEM((C1, 128), jnp.int32),
            pltpu.VMEM((C1, 128), jnp.int32),
            pltpu.VMEM((C1, 128, 16), F32),
            pltpu.VMEM((ZROWS, 16), F32),
            pltpu.VMEM((ZROWS, 16), F32),
            pltpu.SemaphoreType.DMA,
        ],
        compiler_params=_SC_PARAMS,
    )
    o_ss, o_sp, o_ps, o_pp = sc1(xs, xp, ss[0], ss[1], sp[0], sp[1],
                                 ps[0], ps[1], pp[0], pp[1])

    Bb = 2000
    grid = (N // Bb,)
    row16 = pl.BlockSpec((Bb, 16), lambda i: (i, 0))
    w16 = pl.BlockSpec((16, 64), lambda i: (0, 0))
    b64 = pl.BlockSpec((1, 64), lambda i: (0, 0))
    dense1 = pl.pallas_call(
        _dense1_body,
        grid=grid,
        in_specs=[row16] * 5 + [w16] * 4 + [b64] * 2,
        out_specs=[pl.BlockSpec((2, Bb, 32), lambda i: (0, i, 0)),
                   pl.BlockSpec((Bb, 8), lambda i: (i, 0))],
        out_shape=[jax.ShapeDtypeStruct((2, N, 32), F32),
                   jax.ShapeDtypeStruct((N, 8), F32)],
    )
    h_shop, inv_shop = dense1(
        o_ss[:N], o_ss[NPAD:NPAD + N], o_ps[:N], o_ps[NPAD:NPAD + N], xs,
        pad16(p["l1_ss_Wl"]), pad16(p["l1_ps_Wl"]),
        pad16(p["l1_ss_Wr"]), pad16(p["l1_ps_Wr"]),
        p["l1_ss_bl"].reshape(1, 64), p["l1_ps_bl"].reshape(1, 64))
    h_pub, inv_pub = dense1(
        o_sp[:N], o_sp[NPAD:NPAD + N], o_pp[:N], o_pp[NPAD:NPAD + N], xp,
        pad16(p["l1_sp_Wl"]), pad16(p["l1_pp_Wl"]),
        pad16(p["l1_sp_Wr"]), pad16(p["l1_pp_Wr"]),
        p["l1_sp_bl"].reshape(1, 64), p["l1_pp_bl"].reshape(1, 64))

    hcat_shop = h_shop.reshape(2 * N, 32)
    hcat_pub = h_pub.reshape(2 * N, 32)

    sc2 = pl.kernel(
        _sc_l2_body,
        out_type=[jax.ShapeDtypeStruct((NC * NPAD, 32), F32)] * 4,
        mesh=_sc_mesh(),
        scratch_types=[
            pltpu.VMEM_SHARED((NPAD, 32), F32),
            pltpu.VMEM((C2, 128), jnp.int32),
            pltpu.VMEM((C2, 128), jnp.int32),
            pltpu.VMEM((C2, 128, 32), F32),
            pltpu.VMEM((ZROWS, 32), F32),
            pltpu.VMEM((ZROWS, 32), F32),
            pltpu.SemaphoreType.DMA,
        ],
        compiler_params=_SC_PARAMS,
    )
    q_ss, q_sp, q_ps, q_pp = sc2(hcat_shop, hcat_pub, ss[2], ss[1],
                                 sp[2], sp[1], ps[2], ps[1], pp[2], pp[1])

    row32 = pl.BlockSpec((Bb, 32), lambda i: (i, 0))
    w64 = pl.BlockSpec((64, 64), lambda i: (0, 0))
    dense2 = pl.pallas_call(
        _dense2_body,
        grid=grid,
        in_specs=[row32] * 6 + [pl.BlockSpec((Bb, 8), lambda i: (i, 0))]
        + [w64] * 4 + [b64] * 2
        + [pl.BlockSpec((64, 1), lambda i: (0, 0)),
           pl.BlockSpec((1, 1), lambda i: (0, 0))],
        out_specs=pl.BlockSpec((Bb, 1), lambda i: (i, 0)),
        out_shape=jax.ShapeDtypeStruct((N, 1), F32),
    )
    out_shop = dense2(
        q_ss[:N], q_ss[NPAD:NPAD + N], q_ps[:N], q_ps[NPAD:NPAD + N],
        h_shop[0], h_shop[1], inv_shop,
        p["l2_ss_Wl"], p["l2_ps_Wl"], p["l2_ss_Wr"], p["l2_ps_Wr"],
        p["l2_ss_bl"].reshape(1, 64), p["l2_ps_bl"].reshape(1, 64),
        p["lin_shop_W"], p["lin_shop_b"].reshape(1, 1))
    out_public = dense2(
        q_sp[:N], q_sp[NPAD:NPAD + N], q_pp[:N], q_pp[NPAD:NPAD + N],
        h_pub[0], h_pub[1], inv_pub,
        p["l2_sp_Wl"], p["l2_pp_Wl"], p["l2_sp_Wr"], p["l2_pp_Wr"],
        p["l2_sp_bl"].reshape(1, 64), p["l2_pp_bl"].reshape(1, 64),
        p["lin_public_W"], p["lin_public_b"].reshape(1, 1))

    return (out_shop, out_public)


# speedup vs baseline: 7.9041x; 1.2421x over previous
import jax
import jax.numpy as jnp
from jax import lax
from jax.experimental import pallas as pl
from jax.experimental.pallas import tpu as pltpu, tpu_sc as plsc

N = 50000
E = 800000
F32 = jnp.float32

NC, NS = 2, 16
NPAD = 50048
RPT = NPAD // NS
ZROWS = 136
EPAD = 819200
NROWS = EPAD // 128
C1 = 10
C2 = 2
L1_ROWS = NROWS // (NC * NS)
L2_ROWS = NROWS // NS


def _sc_mesh():
    return plsc.VectorSubcoreMesh(
        core_axis_name="c", subcore_axis_name="s", num_cores=NC, num_subcores=NS
    )


_SC_PARAMS = pltpu.CompilerParams(use_tc_tiling_on_sc=False)


def _zero_zbuf(zbuf, width):
    @pl.loop(0, ZROWS)
    def _(i):
        for w in range(width // 16):
            zbuf[i, pl.ds(w * 16, 16)] = jnp.zeros((16,), F32)


def _segment_pass(tbl, src2d, dst2d, out, acc, idxs, idxd, rows, zbuf,
                  bounce, gsem, ssem, c, s, base_rows, cc):
    my_acc_row = s * RPT

    @pl.loop(0, RPT // ZROWS)
    def _(i):
        pltpu.sync_copy(zbuf, acc.at[pl.ds(my_acc_row + i * ZROWS, ZROWS)])

    plsc.subcore_barrier()

    base_s, base_d, tile_rows = base_rows
    half = tile_rows // cc // 2

    def load_idx(b, k):
        e = cc * 128
        pltpu.sync_copy(src2d.at[pl.ds((base_s + k * cc) * 128, e)], idxs.at[b])
        pltpu.sync_copy(dst2d.at[pl.ds((base_d + k * cc) * 128, e)], idxd.at[b])

    def g_start(b):
        pltpu.make_async_copy(tbl.at[idxs.at[b]], rows.at[b], gsem).start()

    def g_wait(b):
        pltpu.make_async_copy(tbl.at[idxs.at[b]], rows.at[b], gsem).wait()

    def s_start(b):
        pltpu.make_async_copy(rows.at[b], acc.at[idxd.at[b]],
                              ssem).start(add=True)

    def s_wait(b):
        pltpu.make_async_copy(rows.at[b], acc.at[idxd.at[b]], ssem).wait()

    @pl.loop(0, half)
    def _(kk):
        for b in (0, 1):
            @pl.when(kk > 0)
            def _():
                s_wait(b)

            load_idx(b, kk * 2 + b)
            g_start(b)
            if b == 0:
                @pl.when(kk > 0)
                def _():
                    g_wait(1)
                    s_start(1)
            else:
                g_wait(0)
                s_start(0)

    g_wait(1)
    s_start(1)
    s_wait(0)
    s_wait(1)

    plsc.subcore_barrier()

    @pl.loop(0, RPT // ZROWS)
    def _(i):
        r = my_acc_row + i * ZROWS
        pltpu.sync_copy(acc.at[pl.ds(r, ZROWS)], bounce)
        pltpu.sync_copy(bounce, out.at[pl.ds(c * NPAD + r, ZROWS)])

    plsc.subcore_barrier()


def _sc_l1_body(xs, xp, s_ss, d_ss, s_sp, d_sp, s_ps, d_ps, s_pp, d_pp,
                o_ss, o_sp, o_ps, o_pp, acc, idxs, idxd, rows, zbuf, bounce,
                gsem, ssem):
    c = lax.axis_index("c")
    s = lax.axis_index("s")
    _zero_zbuf(zbuf, 16)
    base = c * (NROWS // NC) + s * L1_ROWS
    for tbl, src2d, dst2d, out in (
        (xs, s_ss, d_ss, o_ss), (xs, s_sp, d_sp, o_sp),
        (xp, s_ps, d_ps, o_ps), (xp, s_pp, d_pp, o_pp),
    ):
        _segment_pass(tbl, src2d, dst2d, out, acc, idxs, idxd, rows, zbuf,
                      bounce, gsem, ssem, c, s, (base, base, L1_ROWS), C1)


def _sc_l2_body(hs, hp, s_ss, d_ss, s_sp, d_sp, s_ps, d_ps, s_pp, d_pp,
                o_ss, o_sp, o_ps, o_pp, acc, idxs, idxd, rows, zbuf, bounce,
                gsem, ssem):
    c = lax.axis_index("c")
    s = lax.axis_index("s")
    _zero_zbuf(zbuf, 32)
    base_s = c * NROWS + s * L2_ROWS
    base_d = s * L2_ROWS
    for tbl, src2d, dst2d, out in (
        (hs, s_ss, d_ss, o_ss), (hs, s_sp, d_sp, o_sp),
        (hp, s_ps, d_ps, o_ps), (hp, s_pp, d_pp, o_pp),
    ):
        _segment_pass(tbl, src2d, dst2d, out, acc, idxs, idxd, rows, zbuf,
                      bounce, gsem, ssem, c, s, (base_s, base_d, L2_ROWS), C2)


def _dense1_body(p0a, p1a, p0b, p1b, x, wla, wlb, wra, wrb, bla, blb,
                 h_out, inv_out):
    sa = p0a[...] + p1a[...]
    sb = p0b[...] + p1b[...]
    inva = 1.0 / jnp.maximum(sa[:, 11:12], 1.0)
    invb = 1.0 / jnp.maximum(sb[:, 11:12], 1.0)
    ma = sa * inva
    mb = sb * invb
    h = (jnp.dot(ma, wla[...], preferred_element_type=F32)
         + jnp.dot(mb, wlb[...], preferred_element_type=F32)
         + jnp.dot(x[...], wra[...] + wrb[...], preferred_element_type=F32)
         + bla[...] + blb[...])
    h = jnp.maximum(h, 0.0)
    h_out[0] = h[:, :32]
    h_out[1] = h[:, 32:]
    inv_out[...] = jnp.concatenate(
        [inva, invb, jnp.zeros((inva.shape[0], 6), F32)], axis=1)


def _dense2_body(sa0, sa1, sb0, sb1, h0, h1, inv8, wla, wlb, wra, wrb,
                 bla, blb, lw, lb, out):
    ma = jnp.concatenate([sa0[...], sa1[...]], axis=1) * inv8[:, 0:1]
    mb = jnp.concatenate([sb0[...], sb1[...]], axis=1) * inv8[:, 1:2]
    hh = jnp.concatenate([h0[...], h1[...]], axis=1)
    h2 = (jnp.dot(ma, wla[...], preferred_element_type=F32)
          + jnp.dot(mb, wlb[...], preferred_element_type=F32)
          + jnp.dot(hh, wra[...] + wrb[...], preferred_element_type=F32)
          + bla[...] + blb[...])
    h2 = jnp.maximum(h2, 0.0)
    out[...] = jnp.dot(h2, lw[...], preferred_element_type=F32) + lb[...]


def _sc_scratch(cc, width):
    return [
        pltpu.VMEM_SHARED((NPAD, width), F32),
        pltpu.VMEM((2, cc * 128), jnp.int32),
        pltpu.VMEM((2, cc * 128), jnp.int32),
        pltpu.VMEM((2, cc * 128, width), F32),
        pltpu.VMEM((ZROWS, width), F32),
        pltpu.VMEM((ZROWS, width), F32),
        pltpu.SemaphoreType.DMA,
        pltpu.SemaphoreType.DMA,
    ]


def _make_sc1():
    return pl.kernel(
        _sc_l1_body,
        out_type=[jax.ShapeDtypeStruct((NC * NPAD, 16), F32)] * 4,
        mesh=_sc_mesh(),
        scratch_types=_sc_scratch(C1, 16),
        compiler_params=_SC_PARAMS,
    )


def _make_sc2():
    return pl.kernel(
        _sc_l2_body,
        out_type=[jax.ShapeDtypeStruct((NC * NPAD, 32), F32)] * 4,
        mesh=_sc_mesh(),
        scratch_types=_sc_scratch(C2, 32),
        compiler_params=_SC_PARAMS,
    )


def _prep_edges(ei):
    src = jnp.concatenate([ei[0], jnp.zeros((EPAD - E,), jnp.int32)])
    dst = jnp.concatenate([ei[1], jnp.full((EPAD - E,), N, jnp.int32)])
    src_l2 = jnp.concatenate([src, src + N])
    return src, dst, src_l2


def kernel(x_shop, x_public, edge_index_ss, edge_index_sp, edge_index_ps,
           edge_index_pp, params):
    one = jnp.ones((N, 1), F32)
    zpad = jnp.zeros((N, 4), F32)
    xs = jnp.concatenate([x_shop, one, zpad], axis=1)
    xp = jnp.concatenate([x_public, one, zpad], axis=1)

    ss = _prep_edges(edge_index_ss)
    sp = _prep_edges(edge_index_sp)
    ps = _prep_edges(edge_index_ps)
    pp = _prep_edges(edge_index_pp)

    wz = jnp.zeros((5, 64), F32)
    pad16 = lambda w: jnp.concatenate([w, wz], axis=0)
    p = params

    sc1 = _make_sc1()
    o_ss, o_sp, o_ps, o_pp = sc1(xs, xp, ss[0], ss[1], sp[0], sp[1],
                                 ps[0], ps[1], pp[0], pp[1])

    Bb = 2000
    grid = (N // Bb,)
    row16 = pl.BlockSpec((Bb, 16), lambda i: (i, 0))
    w16 = pl.BlockSpec((16, 64), lambda i: (0, 0))
    b64 = pl.BlockSpec((1, 64), lambda i: (0, 0))
    dense1 = pl.pallas_call(
        _dense1_body,
        grid=grid,
        in_specs=[row16] * 5 + [w16] * 4 + [b64] * 2,
        out_specs=[pl.BlockSpec((2, Bb, 32), lambda i: (0, i, 0)),
                   pl.BlockSpec((Bb, 8), lambda i: (i, 0))],
        out_shape=[jax.ShapeDtypeStruct((2, N, 32), F32),
                   jax.ShapeDtypeStruct((N, 8), F32)],
    )
    h_shop, inv_shop = dense1(
        o_ss[:N], o_ss[NPAD:NPAD + N], o_ps[:N], o_ps[NPAD:NPAD + N], xs,
        pad16(p["l1_ss_Wl"]), pad16(p["l1_ps_Wl"]),
        pad16(p["l1_ss_Wr"]), pad16(p["l1_ps_Wr"]),
        p["l1_ss_bl"].reshape(1, 64), p["l1_ps_bl"].reshape(1, 64))
    h_pub, inv_pub = dense1(
        o_sp[:N], o_sp[NPAD:NPAD + N], o_pp[:N], o_pp[NPAD:NPAD + N], xp,
        pad16(p["l1_sp_Wl"]), pad16(p["l1_pp_Wl"]),
        pad16(p["l1_sp_Wr"]), pad16(p["l1_pp_Wr"]),
        p["l1_sp_bl"].reshape(1, 64), p["l1_pp_bl"].reshape(1, 64))

    hcat_shop = h_shop.reshape(2 * N, 32)
    hcat_pub = h_pub.reshape(2 * N, 32)

    sc2 = _make_sc2()
    q_ss, q_sp, q_ps, q_pp = sc2(hcat_shop, hcat_pub, ss[2], ss[1],
                                 sp[2], sp[1], ps[2], ps[1], pp[2], pp[1])

    row32 = pl.BlockSpec((Bb, 32), lambda i: (i, 0))
    w64 = pl.BlockSpec((64, 64), lambda i: (0, 0))
    dense2 = pl.pallas_call(
        _dense2_body,
        grid=grid,
        in_specs=[row32] * 6 + [pl.BlockSpec((Bb, 8), lambda i: (i, 0))]
        + [w64] * 4 + [b64] * 2
        + [pl.BlockSpec((64, 1), lambda i: (0, 0)),
           pl.BlockSpec((1, 1), lambda i: (0, 0))],
        out_specs=pl.BlockSpec((Bb, 1), lambda i: (i, 0)),
        out_shape=jax.ShapeDtypeStruct((N, 1), F32),
    )
    out_shop = dense2(
        q_ss[:N], q_ss[NPAD:NPAD + N], q_ps[:N], q_ps[NPAD:NPAD + N],
        h_shop[0], h_shop[1], inv_shop,
        p["l2_ss_Wl"], p["l2_ps_Wl"], p["l2_ss_Wr"], p["l2_ps_Wr"],
        p["l2_ss_bl"].reshape(1, 64), p["l2_ps_bl"].reshape(1, 64),
        p["lin_shop_W"], p["lin_shop_b"].reshape(1, 1))
    out_public = dense2(
        q_sp[:N], q_sp[NPAD:NPAD + N], q_pp[:N], q_pp[NPAD:NPAD + N],
        h_pub[0], h_pub[1], inv_pub,
        p["l2_sp_Wl"], p["l2_pp_Wl"], p["l2_sp_Wr"], p["l2_pp_Wr"],
        p["l2_sp_bl"].reshape(1, 64), p["l2_pp_bl"].reshape(1, 64),
        p["lin_public_W"], p["lin_public_b"].reshape(1, 1))

    return (out_shop, out_public)
